# Initial kernel scaffold; baseline (speedup 1.0000x reference)
import jax
import jax.numpy as jnp
from jax.experimental import pallas as pl


def kernel(node_rep, node2graph, W1, b1, W2, b2, W3, b3, W4, b4):
    raise NotImplementedError("write your pallas kernel here")


# speedup vs baseline: 3.3126x; 3.3126x over previous
import functools

import jax
import jax.numpy as jnp
from jax.experimental import pallas as pl
from jax.experimental.pallas import tpu as pltpu

H = 128
G = 2048
W = 512


def _softplus(x):
    return jnp.maximum(x, 0.0) + jnp.log(1.0 + jnp.exp(-jnp.abs(x)))


def _body(n_total, nb, b,
          g0_ref, g1_ref, node_ref, ids_ref, W3_ref, b3_ref, W4_ref, b4_ref,
          out_ref, A, P, Pn):
    i = pl.program_id(0)

    @pl.when(i == 0)
    def _init():
        A[...] = jnp.zeros_like(A)
        out_ref[0, 0] = 0.0

    blk = jnp.where(i < nb, i, i - nb - 1)
    g0 = g0_ref[blk]
    g1 = g1_ref[blk]
    g0a = (g0 // 8) * 8

    def one_hot_t(base, prev_end):
        ids2 = ids_ref[0]
        idsb = jnp.broadcast_to(ids2, (W, b))
        gcol = jax.lax.broadcasted_iota(jnp.int32, (W, b), 0) + base
        hit = (idsb == gcol) & (idsb >= prev_end)
        return hit.astype(jnp.bfloat16)

    @pl.when(i < nb)
    def _phase1():
        nwin = (g1 - g0a) // W + 1
        node = node_ref[...].astype(jnp.bfloat16)
        ext = jnp.concatenate([node, jnp.ones((b, H), jnp.bfloat16)], axis=1)

        def win(w, prev_end):
            base = jnp.minimum(g0a + w * W, G - W)
            base = pl.multiple_of(base, 8)
            oht = one_hot_t(base, prev_end)
            part = jax.lax.dot_general(
                oht, ext, (((1,), (0,)), ((), ())),
                preferred_element_type=jnp.float32)
            A[pl.ds(base, W), :] += part
            return base + W

        jax.lax.fori_loop(0, nwin, win, jnp.int32(0))

    @pl.when(i == nb)
    def _phase2():
        seg = A[:, :H]
        cnt = A[:, H:]
        graph_rep = seg / jnp.maximum(cnt, 1.0)
        h = graph_rep @ W3_ref[...] + b3_ref[...]
        h = h * jax.nn.sigmoid(h)
        proj = h @ W4_ref[...] + b4_ref[...]
        P[...] = proj
        Pn[...] = jnp.concatenate([proj[1:], proj[:1]], axis=0)

    @pl.when(i > nb)
    def _phase3():
        nwin = (g1 - g0a) // W + 1
        node = node_ref[...]

        def win(w, carry):
            prev_end, gp, gn = carry
            base = jnp.minimum(g0a + w * W, G - W)
            base = pl.multiple_of(base, 8)
            oht = one_hot_t(base, prev_end)
            Pw = P[pl.ds(base, W), :].astype(jnp.bfloat16)
            Pnw = Pn[pl.ds(base, W), :].astype(jnp.bfloat16)
            gp = gp + jax.lax.dot_general(
                oht, Pw, (((0,), (0,)), ((), ())),
                preferred_element_type=jnp.float32)
            gn = gn + jax.lax.dot_general(
                oht, Pnw, (((0,), (0,)), ((), ())),
                preferred_element_type=jnp.float32)
            return base + W, gp, gn

        z = jnp.zeros((b, H), jnp.float32)
        _, gp, gn = jax.lax.fori_loop(0, nwin, win, (jnp.int32(0), z, z))
        pos = jnp.sum(node * gp, axis=1)
        neg = jnp.sum(node * gn, axis=1)
        s = jnp.sum(_softplus(-pos) + _softplus(neg))
        tot = out_ref[0, 0] + s
        out_ref[0, 0] = jnp.where(i == 2 * nb, tot / n_total, tot)


def kernel(node_rep, node2graph, W1, b1, W2, b2, W3, b3, W4, b4):
    n = node_rep.shape[0]
    b = 1000
    nb = n // b
    assert nb * b == n
    ids = node2graph.astype(jnp.int32)
    ids3 = ids.reshape(nb, 1, b)
    g0s = ids3[:, 0, 0]
    g1s = ids3[:, 0, b - 1]

    grid = 2 * nb + 1

    def nmap(i):
        return (jnp.where(i < nb, i, jnp.maximum(i - nb - 1, 0)), 0)

    def imap(i):
        return (jnp.where(i < nb, i, jnp.maximum(i - nb - 1, 0)), 0, 0)

    out = pl.pallas_call(
        functools.partial(_body, float(n), nb, b),
        grid=(grid,),
        in_specs=[
            pl.BlockSpec(memory_space=pltpu.SMEM),
            pl.BlockSpec(memory_space=pltpu.SMEM),
            pl.BlockSpec((b, H), nmap),
            pl.BlockSpec((1, 1, b), imap),
            pl.BlockSpec((H, H), lambda i: (0, 0)),
            pl.BlockSpec((1, H), lambda i: (0, 0)),
            pl.BlockSpec((H, H), lambda i: (0, 0)),
            pl.BlockSpec((1, H), lambda i: (0, 0)),
        ],
        out_specs=pl.BlockSpec(memory_space=pltpu.SMEM),
        out_shape=jax.ShapeDtypeStruct((1, 1), jnp.float32),
        scratch_shapes=[
            pltpu.VMEM((G, 2 * H), jnp.float32),
            pltpu.VMEM((G, H), jnp.float32),
            pltpu.VMEM((G, H), jnp.float32),
        ],
    )(g0s, g1s, node_rep, ids3, W3, b3.reshape(1, H), W4, b4.reshape(1, H))
    return out[0, 0]
